# baseline (device time: 30920 ns/iter reference)
import os

import jax
import jax.numpy as jnp
from jax import lax
from jax.experimental import pallas as pl
from jax.experimental.pallas import tpu as pltpu

N_DEV = 4
B, SQ, SKV, D = 2, 256, 512, 768
H, DH = 8, 64
R = B * SQ
HALF, QTR = R // 2, R // 4
_PROBE = os.environ.get("KPROBE", "")


def kernel(x, Wq, Wo, K_ext, V_ext):
    def body(x_ref, wq_ref, wo_ref, k_ref, v_ref, out_ref,
             acc_ref, attn_ref, sbuf_a, rbuf_a, sbuf_b, rbuf_b,
             sbuf_q, gbuf, send_sems, recv_sems,
             ag_send_sems, ag_recv_sems):
        my = lax.axis_index("i")
        peer_a = my ^ 1
        peer_b = 3 - my

        barrier_sem = pltpu.get_barrier_semaphore()
        for nbr in (peer_a, peer_b):
            pl.semaphore_signal(
                barrier_sem, inc=1,
                device_id=(nbr,), device_id_type=pl.DeviceIdType.MESH,
            )
        pl.semaphore_wait(barrier_sem, 2)

        is03 = (my == 0) | (my == 3)
        half_keep = jnp.where(is03, 0, HALF)
        half_send = HALF - half_keep
        q_add = jnp.where(my <= 1, 0, QTR)
        q_keep = half_keep + q_add
        q_send = half_keep + (QTR - q_add)

        def compute_batch(b):
            q = jnp.dot(x_ref[b], wq_ref[...],
                        preferred_element_type=jnp.float32)
            for h in range(H):
                qh = q[:, h * DH:(h + 1) * DH]
                kh = k_ref[b, :, h, :]
                vh = v_ref[b, :, h, :]
                s = lax.dot_general(
                    qh, kh, (((1,), (1,)), ((), ())),
                    preferred_element_type=jnp.float32) * 0.125
                m = jnp.max(s, axis=1, keepdims=True)
                p = jnp.exp(s - m)
                l = jnp.sum(p, axis=1, keepdims=True)
                o = jnp.dot(p, vh, preferred_element_type=jnp.float32) / l
                attn_ref[:, h * DH:(h + 1) * DH] = o
            acc_ref[b * SQ:(b + 1) * SQ, :] = jnp.dot(
                attn_ref[...], wo_ref[...],
                preferred_element_type=jnp.float32)

        send_b1 = half_send == HALF
        if _PROBE != "comm":
            pl.when(send_b1)(lambda: compute_batch(1))
            pl.when(jnp.logical_not(send_b1))(lambda: compute_batch(0))

        if _PROBE != "compute":
            sbuf_a[...] = acc_ref[pl.ds(half_send, HALF)].astype(jnp.bfloat16)
            rdma_a = pltpu.make_async_remote_copy(
                src_ref=sbuf_a, dst_ref=rbuf_a,
                send_sem=send_sems.at[0], recv_sem=recv_sems.at[0],
                device_id=(peer_a,), device_id_type=pl.DeviceIdType.MESH,
            )
            rdma_a.start()

        if _PROBE != "comm":
            pl.when(send_b1)(lambda: compute_batch(0))
            pl.when(jnp.logical_not(send_b1))(lambda: compute_batch(1))

        def xchg(src, dst, idx, peer):
            rdma = pltpu.make_async_remote_copy(
                src_ref=src, dst_ref=dst,
                send_sem=send_sems.at[idx], recv_sem=recv_sems.at[idx],
                device_id=(peer,), device_id_type=pl.DeviceIdType.MESH,
            )
            rdma.start()
            rdma.wait()


        if _PROBE != "compute":
            rdma_a.wait()
            acc_ref[pl.ds(half_keep, HALF)] = (
                acc_ref[pl.ds(half_keep, HALF)]
                + rbuf_a[...].astype(jnp.float32))

            sbuf_b[...] = acc_ref[pl.ds(q_send, QTR)].astype(jnp.bfloat16)
            xchg(sbuf_b, rbuf_b, 1, peer_b)
            acc_ref[pl.ds(q_keep, QTR)] = (
                acc_ref[pl.ds(q_keep, QTR)]
                + rbuf_b[...].astype(jnp.float32))

            def qkeep_of(d):
                hk = jnp.where((d == 0) | (d == 3), 0, HALF)
                return hk + jnp.where(d <= 1, 0, QTR)

            def out_write(rows_lo, val):
                b_i = jnp.where(rows_lo >= SQ, 1, 0)
                r0 = rows_lo - b_i * SQ
                out_ref[pl.ds(b_i, 1), pl.ds(r0, QTR), :] = (
                    val.reshape(1, QTR, D))

            sbuf_q[...] = acc_ref[pl.ds(q_keep, QTR)].astype(jnp.bfloat16)
            sends = []
            for r in range(3):
                peer_r = lax.rem(my + 1 + r, N_DEV)
                rdma = pltpu.make_async_remote_copy(
                    src_ref=sbuf_q, dst_ref=gbuf.at[2 - r],
                    send_sem=ag_send_sems.at[r],
                    recv_sem=ag_recv_sems.at[2 - r],
                    device_id=(peer_r,), device_id_type=pl.DeviceIdType.MESH,
                )
                rdma.start()
                sends.append(rdma)

            out_write(q_keep, acc_ref[pl.ds(q_keep, QTR)])

            for s in range(3):
                recv = pltpu.make_async_remote_copy(
                    src_ref=gbuf.at[s], dst_ref=gbuf.at[s],
                    send_sem=ag_send_sems.at[s],
                    recv_sem=ag_recv_sems.at[s],
                    device_id=(my,), device_id_type=pl.DeviceIdType.MESH,
                )
                recv.wait_recv()
                out_write(qkeep_of(lax.rem(my + 1 + s, N_DEV)),
                          gbuf[s].astype(jnp.float32))

            for rdma in sends:
                rdma.wait_send()

    return pl.pallas_call(
        body,
        out_shape=jax.ShapeDtypeStruct((B, SQ, D), jnp.float32),
        in_specs=[pl.BlockSpec(memory_space=pltpu.VMEM)] * 5,
        out_specs=pl.BlockSpec(memory_space=pltpu.VMEM),
        scratch_shapes=[
            pltpu.VMEM((R, D), jnp.float32),
            pltpu.VMEM((SQ, H * DH), jnp.float32),
            pltpu.VMEM((HALF, D), jnp.bfloat16),
            pltpu.VMEM((HALF, D), jnp.bfloat16),
            pltpu.VMEM((QTR, D), jnp.bfloat16),
            pltpu.VMEM((QTR, D), jnp.bfloat16),
            pltpu.VMEM((QTR, D), jnp.bfloat16),
            pltpu.VMEM((3, QTR, D), jnp.bfloat16),
            pltpu.SemaphoreType.DMA((2,)),
            pltpu.SemaphoreType.DMA((2,)),
            pltpu.SemaphoreType.DMA((3,)),
            pltpu.SemaphoreType.DMA((3,)),
        ],
        compiler_params=pltpu.CompilerParams(collective_id=0),
    )(x, Wq, Wo, K_ext, V_ext)


# device time: 28238 ns/iter; 1.0950x vs baseline; 1.0950x over previous
import os

import jax
import jax.numpy as jnp
from jax import lax
from jax.experimental import pallas as pl
from jax.experimental.pallas import tpu as pltpu

N_DEV = 4
B, SQ, SKV, D = 2, 256, 512, 768
H, DH = 8, 64
R = B * SQ
HALF, QTR = R // 2, R // 4
_PROBE = os.environ.get("KPROBE", "")


def kernel(x, Wq, Wo, K_ext, V_ext):
    def body(x_ref, wq_ref, wo_ref, k_ref, v_ref, out_ref,
             acc_ref, attn_ref, sbuf_a, rbuf_a, sbuf_b, rbuf_b,
             sbuf_q, gbuf, send_sems, recv_sems,
             ag_send_sems, ag_recv_sems):
        my = lax.axis_index("i")
        peer_a = my ^ 1
        peer_b = 3 - my

        barrier_sem = pltpu.get_barrier_semaphore()
        for nbr in (peer_a, peer_b):
            pl.semaphore_signal(
                barrier_sem, inc=1,
                device_id=(nbr,), device_id_type=pl.DeviceIdType.MESH,
            )
        pl.semaphore_wait(barrier_sem, 2)

        is03 = (my == 0) | (my == 3)
        half_keep = jnp.where(is03, 0, HALF)
        half_send = HALF - half_keep
        q_add = jnp.where(my <= 1, 0, QTR)
        q_keep = half_keep + q_add
        q_send = half_keep + (QTR - q_add)

        def compute_batch(b):
            q = jnp.dot(x_ref[b], wq_ref[...],
                        preferred_element_type=jnp.float32)
            for h in range(H):
                qh = q[:, h * DH:(h + 1) * DH]
                kt = k_ref[b, h]
                vh = v_ref[b, h]
                s = jnp.dot(qh, kt,
                            preferred_element_type=jnp.float32) * 0.125
                m = jnp.max(s, axis=1, keepdims=True)
                p = jnp.exp(s - m)
                l = jnp.sum(p, axis=1, keepdims=True)
                o = jnp.dot(p, vh, preferred_element_type=jnp.float32) / l
                attn_ref[:, h * DH:(h + 1) * DH] = o
            acc_ref[b * SQ:(b + 1) * SQ, :] = jnp.dot(
                attn_ref[...], wo_ref[...],
                preferred_element_type=jnp.float32)

        send_b1 = half_send == HALF
        if _PROBE != "comm":
            pl.when(send_b1)(lambda: compute_batch(1))
            pl.when(jnp.logical_not(send_b1))(lambda: compute_batch(0))

        if _PROBE != "compute":
            sbuf_a[...] = acc_ref[pl.ds(half_send, HALF)].astype(jnp.bfloat16)
            rdma_a = pltpu.make_async_remote_copy(
                src_ref=sbuf_a, dst_ref=rbuf_a,
                send_sem=send_sems.at[0], recv_sem=recv_sems.at[0],
                device_id=(peer_a,), device_id_type=pl.DeviceIdType.MESH,
            )
            rdma_a.start()

        if _PROBE != "comm":
            pl.when(send_b1)(lambda: compute_batch(0))
            pl.when(jnp.logical_not(send_b1))(lambda: compute_batch(1))

        def xchg(src, dst, idx, peer):
            rdma = pltpu.make_async_remote_copy(
                src_ref=src, dst_ref=dst,
                send_sem=send_sems.at[idx], recv_sem=recv_sems.at[idx],
                device_id=(peer,), device_id_type=pl.DeviceIdType.MESH,
            )
            rdma.start()
            rdma.wait()


        if _PROBE != "compute":
            rdma_a.wait()
            acc_ref[pl.ds(half_keep, HALF)] = (
                acc_ref[pl.ds(half_keep, HALF)]
                + rbuf_a[...].astype(jnp.float32))

            sbuf_b[...] = acc_ref[pl.ds(q_send, QTR)].astype(jnp.bfloat16)
            xchg(sbuf_b, rbuf_b, 1, peer_b)
            acc_ref[pl.ds(q_keep, QTR)] = (
                acc_ref[pl.ds(q_keep, QTR)]
                + rbuf_b[...].astype(jnp.float32))

            def qkeep_of(d):
                hk = jnp.where((d == 0) | (d == 3), 0, HALF)
                return hk + jnp.where(d <= 1, 0, QTR)

            def out_write(rows_lo, val):
                b_i = jnp.where(rows_lo >= SQ, 1, 0)
                r0 = rows_lo - b_i * SQ
                out_ref[pl.ds(b_i, 1), pl.ds(r0, QTR), :] = (
                    val.reshape(1, QTR, D))

            sbuf_q[...] = acc_ref[pl.ds(q_keep, QTR)].astype(jnp.bfloat16)
            sends = []
            for r in range(3):
                peer_r = lax.rem(my + 1 + r, N_DEV)
                rdma = pltpu.make_async_remote_copy(
                    src_ref=sbuf_q, dst_ref=gbuf.at[2 - r],
                    send_sem=ag_send_sems.at[r],
                    recv_sem=ag_recv_sems.at[2 - r],
                    device_id=(peer_r,), device_id_type=pl.DeviceIdType.MESH,
                )
                rdma.start()
                sends.append(rdma)

            out_write(q_keep, acc_ref[pl.ds(q_keep, QTR)])

            for s in range(3):
                recv = pltpu.make_async_remote_copy(
                    src_ref=gbuf.at[s], dst_ref=gbuf.at[s],
                    send_sem=ag_send_sems.at[s],
                    recv_sem=ag_recv_sems.at[s],
                    device_id=(my,), device_id_type=pl.DeviceIdType.MESH,
                )
                recv.wait_recv()
                out_write(qkeep_of(lax.rem(my + 1 + s, N_DEV)),
                          gbuf[s].astype(jnp.float32))

            for rdma in sends:
                rdma.wait_send()

    return pl.pallas_call(
        body,
        out_shape=jax.ShapeDtypeStruct((B, SQ, D), jnp.float32),
        in_specs=[pl.BlockSpec(memory_space=pltpu.VMEM)] * 5,
        out_specs=pl.BlockSpec(memory_space=pltpu.VMEM),
        scratch_shapes=[
            pltpu.VMEM((R, D), jnp.float32),
            pltpu.VMEM((SQ, H * DH), jnp.float32),
            pltpu.VMEM((HALF, D), jnp.bfloat16),
            pltpu.VMEM((HALF, D), jnp.bfloat16),
            pltpu.VMEM((QTR, D), jnp.bfloat16),
            pltpu.VMEM((QTR, D), jnp.bfloat16),
            pltpu.VMEM((QTR, D), jnp.bfloat16),
            pltpu.VMEM((3, QTR, D), jnp.bfloat16),
            pltpu.SemaphoreType.DMA((2,)),
            pltpu.SemaphoreType.DMA((2,)),
            pltpu.SemaphoreType.DMA((3,)),
            pltpu.SemaphoreType.DMA((3,)),
        ],
        compiler_params=pltpu.CompilerParams(collective_id=0),
    )(x, Wq, Wo,
      jnp.transpose(K_ext, (0, 2, 3, 1)),
      jnp.transpose(V_ext, (0, 2, 1, 3)))


# device time: 25623 ns/iter; 1.2067x vs baseline; 1.1021x over previous
import os

import jax
import jax.numpy as jnp
from jax import lax
from jax.experimental import pallas as pl
from jax.experimental.pallas import tpu as pltpu

N_DEV = 4
B, SQ, SKV, D = 2, 256, 512, 768
H, DH = 8, 64
R = B * SQ
QTR = R // 4
_PROBE = os.environ.get("KPROBE", "")


def kernel(x, Wq, Wo, K_ext, V_ext):
    def body(x_ref, wq_ref, wo_ref, k_ref, v_ref, out_ref,
             acc_ref, attn_ref, rsend, rrecv, sbuf_q, gbuf,
             rs_send_sems, rs_recv_sems, ag_send_sems, ag_recv_sems):
        my = lax.axis_index("i")
        peers = [lax.rem(my + 1 + r, N_DEV) for r in range(3)]

        def qkeep_of(d):
            hk = jnp.where((d == 0) | (d == 3), 0, 2 * QTR)
            return hk + jnp.where(d <= 1, 0, QTR)

        q_keep = qkeep_of(my)
        qk_r = [qkeep_of(peers[r]) for r in range(3)]

        barrier_sem = pltpu.get_barrier_semaphore()
        for r in range(3):
            pl.semaphore_signal(
                barrier_sem, inc=1,
                device_id=(peers[r],), device_id_type=pl.DeviceIdType.MESH,
            )
        pl.semaphore_wait(barrier_sem, 3)

        def compute_batch(b):
            q = jnp.dot(x_ref[b], wq_ref[...],
                        preferred_element_type=jnp.float32)
            for h in range(H):
                qh = q[:, h * DH:(h + 1) * DH]
                kt = k_ref[b, h]
                vh = v_ref[b, h]
                s = jnp.dot(qh, kt,
                            preferred_element_type=jnp.float32) * 0.125
                m = jnp.max(s, axis=1, keepdims=True)
                p = jnp.exp(s - m)
                l = jnp.sum(p, axis=1, keepdims=True)
                o = jnp.dot(p, vh, preferred_element_type=jnp.float32) / l
                attn_ref[:, h * DH:(h + 1) * DH] = o
            acc_ref[b * SQ:(b + 1) * SQ, :] = jnp.dot(
                attn_ref[...], wo_ref[...],
                preferred_element_type=jnp.float32)

        rs_sends = [
            pltpu.make_async_remote_copy(
                src_ref=rsend.at[r], dst_ref=rrecv.at[2 - r],
                send_sem=rs_send_sems.at[r], recv_sem=rs_recv_sems.at[2 - r],
                device_id=(peers[r],), device_id_type=pl.DeviceIdType.MESH,
            )
            for r in range(3)
        ]

        def rs_send_phase(bb):
            for r in range(3):
                def _go(r=r):
                    rsend[r] = acc_ref[pl.ds(qk_r[r], QTR)].astype(
                        jnp.bfloat16)
                    rs_sends[r].start()
                pl.when((qk_r[r] >= SQ) == (bb == 1))(_go)

        first_b1 = (my == 0) | (my == 3)
        first_b = jnp.where(first_b1, 1, 0)

        if _PROBE != "comm":
            pl.when(first_b1)(lambda: compute_batch(1))
            pl.when(jnp.logical_not(first_b1))(lambda: compute_batch(0))
        if _PROBE != "compute":
            rs_send_phase(first_b)
        if _PROBE != "comm":
            pl.when(first_b1)(lambda: compute_batch(0))
            pl.when(jnp.logical_not(first_b1))(lambda: compute_batch(1))

        if _PROBE != "compute":
            rs_send_phase(1 - first_b)

            for s in range(3):
                recv = pltpu.make_async_remote_copy(
                    src_ref=rrecv.at[s], dst_ref=rrecv.at[s],
                    send_sem=rs_send_sems.at[s],
                    recv_sem=rs_recv_sems.at[s],
                    device_id=(my,), device_id_type=pl.DeviceIdType.MESH,
                )
                recv.wait_recv()
                acc_ref[pl.ds(q_keep, QTR)] = (
                    acc_ref[pl.ds(q_keep, QTR)]
                    + rrecv[s].astype(jnp.float32))

            def out_write(rows_lo, val):
                b_i = jnp.where(rows_lo >= SQ, 1, 0)
                r0 = rows_lo - b_i * SQ
                out_ref[pl.ds(b_i, 1), pl.ds(r0, QTR), :] = (
                    val.reshape(1, QTR, D))

            sbuf_q[...] = acc_ref[pl.ds(q_keep, QTR)].astype(jnp.bfloat16)
            ag_sends = []
            for r in range(3):
                rdma = pltpu.make_async_remote_copy(
                    src_ref=sbuf_q, dst_ref=gbuf.at[2 - r],
                    send_sem=ag_send_sems.at[r],
                    recv_sem=ag_recv_sems.at[2 - r],
                    device_id=(peers[r],), device_id_type=pl.DeviceIdType.MESH,
                )
                rdma.start()
                ag_sends.append(rdma)

            out_write(q_keep, acc_ref[pl.ds(q_keep, QTR)])

            for s in range(3):
                recv = pltpu.make_async_remote_copy(
                    src_ref=gbuf.at[s], dst_ref=gbuf.at[s],
                    send_sem=ag_send_sems.at[s],
                    recv_sem=ag_recv_sems.at[s],
                    device_id=(my,), device_id_type=pl.DeviceIdType.MESH,
                )
                recv.wait_recv()
                out_write(qkeep_of(peers[s]), gbuf[s].astype(jnp.float32))

            for rdma in rs_sends + ag_sends:
                rdma.wait_send()

    return pl.pallas_call(
        body,
        out_shape=jax.ShapeDtypeStruct((B, SQ, D), jnp.float32),
        in_specs=[pl.BlockSpec(memory_space=pltpu.VMEM)] * 5,
        out_specs=pl.BlockSpec(memory_space=pltpu.VMEM),
        scratch_shapes=[
            pltpu.VMEM((R, D), jnp.float32),
            pltpu.VMEM((SQ, H * DH), jnp.float32),
            pltpu.VMEM((3, QTR, D), jnp.bfloat16),
            pltpu.VMEM((3, QTR, D), jnp.bfloat16),
            pltpu.VMEM((QTR, D), jnp.bfloat16),
            pltpu.VMEM((3, QTR, D), jnp.bfloat16),
            pltpu.SemaphoreType.DMA((3,)),
            pltpu.SemaphoreType.DMA((3,)),
            pltpu.SemaphoreType.DMA((3,)),
            pltpu.SemaphoreType.DMA((3,)),
        ],
        compiler_params=pltpu.CompilerParams(collective_id=0),
    )(x, Wq, Wo,
      jnp.transpose(K_ext, (0, 2, 3, 1)),
      jnp.transpose(V_ext, (0, 2, 1, 3)))


# device time: 25291 ns/iter; 1.2226x vs baseline; 1.0131x over previous
import os

import jax
import jax.numpy as jnp
from jax import lax
from jax.experimental import pallas as pl
from jax.experimental.pallas import tpu as pltpu

N_DEV = 4
B, SQ, SKV, D = 2, 256, 512, 768
H, DH = 8, 64
R = B * SQ
QTR = R // 4
_PROBE = os.environ.get("KPROBE", "")


def kernel(x, Wq, Wo, K_ext, V_ext):
    def body(x_ref, wq_ref, wo_ref, k_ref, v_ref, out_ref,
             acc_ref, attn_ref, rsend, rrecv, sbuf_q, gbuf,
             rs_send_sems, rs_recv_sems, ag_send_sems, ag_recv_sems):
        my = lax.axis_index("i")
        peers = [lax.rem(my + 1 + r, N_DEV) for r in range(3)]

        def qkeep_of(d):
            hk = jnp.where((d == 0) | (d == 3), 0, 2 * QTR)
            return hk + jnp.where(d <= 1, 0, QTR)

        q_keep = qkeep_of(my)
        qk_r = [qkeep_of(peers[r]) for r in range(3)]

        barrier_sem = pltpu.get_barrier_semaphore()
        for r in range(3):
            pl.semaphore_signal(
                barrier_sem, inc=1,
                device_id=(peers[r],), device_id_type=pl.DeviceIdType.MESH,
            )
        pl.semaphore_wait(barrier_sem, 3)

        def compute_batch(b):
            q = jnp.dot(x_ref[b], wq_ref[...],
                        preferred_element_type=jnp.float32)
            for h in range(H):
                qh = q[:, h * DH:(h + 1) * DH]
                kt = k_ref[b, h]
                vh = v_ref[b, h]
                s = jnp.dot(qh, kt,
                            preferred_element_type=jnp.float32) * 0.125
                m = jnp.max(s, axis=1, keepdims=True)
                p = jnp.exp(s - m)
                l = jnp.sum(p, axis=1, keepdims=True)
                o = jnp.dot(p, vh, preferred_element_type=jnp.float32) / l
                attn_ref[:, h * DH:(h + 1) * DH] = o
            acc_ref[b * SQ:(b + 1) * SQ, :] = jnp.dot(
                attn_ref[...], wo_ref[...],
                preferred_element_type=jnp.float32)

        def compute_quarter(b, g):
            r0 = g - b * SQ
            xq = x_ref[b, pl.ds(r0, QTR), :]
            q = jnp.dot(xq, wq_ref[...],
                        preferred_element_type=jnp.float32)
            for h in range(H):
                qh = q[:, h * DH:(h + 1) * DH]
                kt = k_ref[b, h]
                vh = v_ref[b, h]
                s = jnp.dot(qh, kt,
                            preferred_element_type=jnp.float32) * 0.125
                m = jnp.max(s, axis=1, keepdims=True)
                p = jnp.exp(s - m)
                l = jnp.sum(p, axis=1, keepdims=True)
                o = jnp.dot(p, vh, preferred_element_type=jnp.float32) / l
                attn_ref[0:QTR, h * DH:(h + 1) * DH] = o
            acc_ref[pl.ds(g, QTR)] = jnp.dot(
                attn_ref[0:QTR, :], wo_ref[...],
                preferred_element_type=jnp.float32)

        rs_sends = [
            pltpu.make_async_remote_copy(
                src_ref=rsend.at[r], dst_ref=rrecv.at[2 - r],
                send_sem=rs_send_sems.at[r], recv_sem=rs_recv_sems.at[2 - r],
                device_id=(peers[r],), device_id_type=pl.DeviceIdType.MESH,
            )
            for r in range(3)
        ]

        def rs_send_phase(bb):
            for r in range(3):
                def _go(r=r):
                    rsend[r] = acc_ref[pl.ds(qk_r[r], QTR)].astype(
                        jnp.bfloat16)
                    rs_sends[r].start()
                pl.when((qk_r[r] >= SQ) == (bb == 1))(_go)

        first_b1 = (my == 0) | (my == 3)
        first_b = jnp.where(first_b1, 1, 0)

        if _PROBE != "comm":
            pl.when(first_b1)(lambda: compute_batch(1))
            pl.when(jnp.logical_not(first_b1))(lambda: compute_batch(0))
        if _PROBE != "compute":
            rs_send_phase(first_b)

        def second_phase(b_s):
            late_g = 2 * b_s * SQ + QTR - q_keep
            if _PROBE != "comm":
                compute_quarter(b_s, late_g)
            if _PROBE != "compute":
                rs_send_phase(1 - first_b)
            if _PROBE != "comm":
                compute_quarter(b_s, q_keep)

        pl.when(first_b1)(lambda: second_phase(0))
        pl.when(jnp.logical_not(first_b1))(lambda: second_phase(1))

        if _PROBE != "compute":
            for s in range(3):
                recv = pltpu.make_async_remote_copy(
                    src_ref=rrecv.at[s], dst_ref=rrecv.at[s],
                    send_sem=rs_send_sems.at[s],
                    recv_sem=rs_recv_sems.at[s],
                    device_id=(my,), device_id_type=pl.DeviceIdType.MESH,
                )
                recv.wait_recv()
                acc_ref[pl.ds(q_keep, QTR)] = (
                    acc_ref[pl.ds(q_keep, QTR)]
                    + rrecv[s].astype(jnp.float32))

            def out_write(rows_lo, val):
                b_i = jnp.where(rows_lo >= SQ, 1, 0)
                r0 = rows_lo - b_i * SQ
                out_ref[pl.ds(b_i, 1), pl.ds(r0, QTR), :] = (
                    val.reshape(1, QTR, D))

            sbuf_q[...] = acc_ref[pl.ds(q_keep, QTR)].astype(jnp.bfloat16)
            ag_sends = []
            for r in range(3):
                rdma = pltpu.make_async_remote_copy(
                    src_ref=sbuf_q, dst_ref=gbuf.at[2 - r],
                    send_sem=ag_send_sems.at[r],
                    recv_sem=ag_recv_sems.at[2 - r],
                    device_id=(peers[r],), device_id_type=pl.DeviceIdType.MESH,
                )
                rdma.start()
                ag_sends.append(rdma)

            out_write(q_keep, acc_ref[pl.ds(q_keep, QTR)])

            for s in range(3):
                recv = pltpu.make_async_remote_copy(
                    src_ref=gbuf.at[s], dst_ref=gbuf.at[s],
                    send_sem=ag_send_sems.at[s],
                    recv_sem=ag_recv_sems.at[s],
                    device_id=(my,), device_id_type=pl.DeviceIdType.MESH,
                )
                recv.wait_recv()
                out_write(qkeep_of(peers[s]), gbuf[s].astype(jnp.float32))

            for rdma in rs_sends + ag_sends:
                rdma.wait_send()

    return pl.pallas_call(
        body,
        out_shape=jax.ShapeDtypeStruct((B, SQ, D), jnp.float32),
        in_specs=[pl.BlockSpec(memory_space=pltpu.VMEM)] * 5,
        out_specs=pl.BlockSpec(memory_space=pltpu.VMEM),
        scratch_shapes=[
            pltpu.VMEM((R, D), jnp.float32),
            pltpu.VMEM((SQ, H * DH), jnp.float32),
            pltpu.VMEM((3, QTR, D), jnp.bfloat16),
            pltpu.VMEM((3, QTR, D), jnp.bfloat16),
            pltpu.VMEM((QTR, D), jnp.bfloat16),
            pltpu.VMEM((3, QTR, D), jnp.bfloat16),
            pltpu.SemaphoreType.DMA((3,)),
            pltpu.SemaphoreType.DMA((3,)),
            pltpu.SemaphoreType.DMA((3,)),
            pltpu.SemaphoreType.DMA((3,)),
        ],
        compiler_params=pltpu.CompilerParams(collective_id=0),
    )(x, Wq, Wo,
      jnp.transpose(K_ext, (0, 2, 3, 1)),
      jnp.transpose(V_ext, (0, 2, 1, 3)))
